# baseline (speedup 1.0000x reference)
import functools

import jax
import jax.numpy as jnp
from jax import lax
from jax.experimental import pallas as pl
from jax.experimental.pallas import tpu as pltpu
from jax.experimental.pallas import tpu_sc as plsc

ALPHA = 4
D = 64
SEQ_LEN = 65536
NC = 2
NS = 16
NW = NC * NS
CHAINS_PER_W = 4
NCHAINS = NW * CHAINS_PER_W
CHAIN_LEN = SEQ_LEN // NCHAINS
W_LEN = CHAINS_PER_W * CHAIN_LEN
REC = 96


def _sc_count(seq_hbm, tab_hbm, out_hbm, seq_v, tab_v, acc_v):
    wid = lax.axis_index("s") * NC + lax.axis_index("c")
    base = wid * W_LEN
    pltpu.sync_copy(seq_hbm.at[pl.ds(base, W_LEN)], seq_v)
    pltpu.sync_copy(tab_hbm, tab_v)

    zeros = jnp.zeros((16,), jnp.float32)
    for j in range(CHAINS_PER_W * REC // 16):
        acc_v[pl.ds(j * 16, 16)] = zeros

    def body(i, carry):
        new = []
        for k in range(CHAINS_PER_W):
            c1e, c2 = carry[2 * k], carry[2 * k + 1]
            xv = seq_v[pl.ds(k * CHAIN_LEN + i * 16, 16)]
            for j in range(16):
                x = xv[j]
                off = k * REC + 32 + x * 16
                acc_v[pl.ds(off, 16)] = acc_v[pl.ds(off, 16)] + c2
                c2 = c2 + c1e * tab_v[pl.ds(x * 16, 16)]
                c1e = c1e + tab_v[pl.ds(64 + x * 16, 16)]
            new += [c1e, c2]
        return tuple(new)

    carry = lax.fori_loop(0, CHAIN_LEN // 16, body,
                          tuple(zeros for _ in range(2 * CHAINS_PER_W)))
    for k in range(CHAINS_PER_W):
        acc_v[pl.ds(k * REC, 16)] = carry[2 * k]
        acc_v[pl.ds(k * REC + 16, 16)] = carry[2 * k + 1]
    pltpu.sync_copy(acc_v, out_hbm.at[pl.ds(wid * CHAINS_PER_W * REC,
                                            CHAINS_PER_W * REC)])


def _bcast_cols(x, k, reps):
    n = x.shape[1]
    return jnp.concatenate(
        [jnp.broadcast_to(x[:, j:j + 1], (x.shape[0], reps)) for j in range(n)],
        axis=1)


def _merge_kernel(part_ref, idx_ref, sgn_ref, cs_ref, chm_ref, dw_ref, bias_ref,
                  w1t_ref, b1_ref, w2t_ref, b2_ref, out_ref):
    Prt = part_ref[:]
    c1e_blk = Prt[:, 0:16]
    c2_blk = Prt[:, 16:32]
    c3_blk = Prt[:, 32:96]

    jr = lax.broadcasted_iota(jnp.int32, (16, ALPHA), 0)
    jc = lax.broadcasted_iota(jnp.int32, (16, ALPHA), 1)
    S = jnp.where(((jr >> 2) == jc) & ((jr & 3) == 0), 1.0, 0.0)
    C1 = jnp.dot(c1e_blk, S, preferred_element_type=jnp.float32)

    mi = lax.broadcasted_iota(jnp.int32, (D, D), 0)
    ki = lax.broadcasted_iota(jnp.int32, (D, D), 1)
    M = jnp.where(ki == (((mi >> 2) & 3) * 16 + (mi & 3) * 4 + (mi >> 4)), 1.0, 0.0)
    C3 = jnp.dot(c3_blk, M, preferred_element_type=jnp.float32)

    il_r = lax.broadcasted_iota(jnp.int32, (NCHAINS, NCHAINS), 0)
    il_c = lax.broadcasted_iota(jnp.int32, (NCHAINS, NCHAINS), 1)
    L = (il_c < il_r).astype(jnp.float32)

    pre1 = jnp.dot(L, C1, preferred_element_type=jnp.float32)
    q2inc = c2_blk + _bcast_cols(pre1, 0, 4) * jnp.concatenate([C1] * 4, axis=1)
    pre2 = jnp.dot(L, q2inc, preferred_element_type=jnp.float32)

    c3contrib = (C3
                 + _bcast_cols(pre2, 0, 4) * jnp.concatenate([C1] * 16, axis=1)
                 + _bcast_cols(pre1, 0, 16) * jnp.concatenate([c2_blk] * 4, axis=1))
    c3row = jnp.sum(c3contrib, axis=0, keepdims=True)
    c1row = jnp.sum(C1, axis=0, keepdims=True)

    ed = lax.broadcasted_iota(jnp.int32, (D, D), 1)
    E = jnp.where(ed == idx_ref[:], sgn_ref[:], 0.0)
    baseline = jnp.dot(c3row, E, preferred_element_type=jnp.float32)

    inv_n = 1.0 / SEQ_LEN
    scaling = jnp.sum(c1row * cs_ref[:], axis=1, keepdims=True) * inv_n
    mods = jnp.dot(c1row, chm_ref[:], preferred_element_type=jnp.float32) * inv_n

    enhanced = (baseline * dw_ref[:] + bias_ref[:]) * scaling + mods
    hidden = jnp.maximum(
        jnp.dot(enhanced, w1t_ref[:], preferred_element_type=jnp.float32) + b1_ref[:], 0.0)
    out_ref[:] = jnp.dot(hidden, w2t_ref[:], preferred_element_type=jnp.float32) + b2_ref[:]


@functools.cache
def _sc_count_call():
    return pl.kernel(
        _sc_count,
        out_type=jax.ShapeDtypeStruct((NCHAINS * REC,), jnp.float32),
        mesh=plsc.VectorSubcoreMesh(core_axis_name="c", subcore_axis_name="s"),
        scratch_types=[
            pltpu.VMEM((W_LEN,), jnp.int32),
            pltpu.VMEM((128,), jnp.float32),
            pltpu.VMEM((CHAINS_PER_W * REC,), jnp.float32),
        ],
    )


def kernel(sequence, h_hash, s_signs, char_scales, dimension_weights, sketch_bias,
           char_hash_modifiers, W1, b1, W2, b2):
    lane = jnp.arange(16, dtype=jnp.int32)
    mcol = (lane[None, :] & 3) == jnp.arange(ALPHA, dtype=jnp.int32)[:, None]
    mrow = (lane[None, :] >> 2) == jnp.arange(ALPHA, dtype=jnp.int32)[:, None]
    tables = jnp.concatenate(
        [mcol.astype(jnp.float32).reshape(64), mrow.astype(jnp.float32).reshape(64)])
    partials = _sc_count_call()(sequence, tables).reshape(NCHAINS, REC)

    idx64 = jnp.reshape(
        (h_hash[0][:, None, None] + h_hash[1][None, :, None] + h_hash[2][None, None, :]) % D,
        (D, 1)).astype(jnp.int32)
    sgn64 = jnp.reshape(
        s_signs[0][:, None, None] * s_signs[1][None, :, None] * s_signs[2][None, None, :],
        (D, 1))

    out = pl.pallas_call(
        _merge_kernel,
        out_shape=jax.ShapeDtypeStruct((1, D), jnp.float32),
    )(partials, idx64, sgn64,
      char_scales.reshape(1, ALPHA), char_hash_modifiers,
      dimension_weights.reshape(1, D), sketch_bias.reshape(1, D),
      W1.T, b1.reshape(1, D), W2.T, b2.reshape(1, D))
    return out.reshape(D)


# speedup vs baseline: 1.1771x; 1.1771x over previous
import functools

import jax
import jax.numpy as jnp
from jax import lax
from jax.experimental import pallas as pl
from jax.experimental.pallas import tpu as pltpu
from jax.experimental.pallas import tpu_sc as plsc

ALPHA = 4
D = 64
SEQ_LEN = 65536
NC = 2
NS = 16
NW = NC * NS
LANES = 16
NCHAINS = NW * LANES
CHAIN_LEN = SEQ_LEN // NCHAINS
W_LEN = LANES * CHAIN_LEN
REC = 96


def _sc_count(seq_hbm, out_hbm, seq_v, c3_v, rec_v):
    wid = lax.axis_index("s") * NC + lax.axis_index("c")
    pltpu.sync_copy(seq_hbm.at[pl.ds(wid * W_LEN, W_LEN)], seq_v)

    zeros = jnp.zeros((16,), jnp.float32)
    for j in range(LANES * D // 16):
        c3_v[pl.ds(j * 16, 16)] = zeros

    def body(_, carry):
        gidx = carry[0]
        C1 = list(carry[1:5])
        C2 = list(carry[5:21])
        lane = lax.iota(jnp.int32, 16)
        xv = plsc.load_gather(seq_v, [gidx])
        base = (xv << 4) + lane
        m = [(xv == b).astype(jnp.float32) for b in range(ALPHA)]
        for ab in range(16):
            plsc.addupdate_scatter(c3_v, [base + (ab * 64)], C2[ab])
        for a in range(ALPHA):
            for b in range(ALPHA):
                C2[a * 4 + b] = C2[a * 4 + b] + C1[a] * m[b]
        for a in range(ALPHA):
            C1[a] = C1[a] + m[a]
        return tuple([gidx + 1] + C1 + C2)

    gidx0 = lax.iota(jnp.int32, 16) * CHAIN_LEN
    carry = lax.fori_loop(0, CHAIN_LEN, body,
                          tuple([gidx0] + [zeros] * 20))
    C1 = carry[1:5]
    C2 = carry[5:21]

    lane = lax.iota(jnp.int32, 16)
    riota = lane * REC
    for a in range(ALPHA):
        plsc.store_scatter(rec_v, [riota + a], C1[a])
    for ab in range(16):
        plsc.store_scatter(rec_v, [riota + (4 + ab)], C2[ab])
    for ab in range(16):
        for c in range(ALPHA):
            v = c3_v[pl.ds(ab * 64 + c * 16, 16)]
            plsc.store_scatter(rec_v, [riota + (20 + ab * 4 + c)], v)
    zpad = jnp.zeros((16,), jnp.float32)
    for p in range(84, REC):
        plsc.store_scatter(rec_v, [riota + p], zpad)
    pltpu.sync_copy(rec_v, out_hbm.at[pl.ds(wid * LANES * REC, LANES * REC)])


def _bcast_cols(x, k, reps):
    n = x.shape[1]
    return jnp.concatenate(
        [jnp.broadcast_to(x[:, j:j + 1], (x.shape[0], reps)) for j in range(n)],
        axis=1)


def _merge_kernel(part_ref, idx_ref, sgn_ref, cs_ref, chm_ref, dw_ref, bias_ref,
                  w1t_ref, b1_ref, w2t_ref, b2_ref, out_ref):
    Prt = part_ref[:]
    C1 = Prt[:, 0:4]
    c2_blk = Prt[:, 4:20]
    C3 = Prt[:, 20:84]

    il_r = lax.broadcasted_iota(jnp.int32, (NCHAINS, NCHAINS), 0)
    il_c = lax.broadcasted_iota(jnp.int32, (NCHAINS, NCHAINS), 1)
    L = (il_c < il_r).astype(jnp.float32)

    pre1 = jnp.dot(L, C1, preferred_element_type=jnp.float32)
    q2inc = c2_blk + _bcast_cols(pre1, 0, 4) * jnp.concatenate([C1] * 4, axis=1)
    pre2 = jnp.dot(L, q2inc, preferred_element_type=jnp.float32)

    c3contrib = (C3
                 + _bcast_cols(pre2, 0, 4) * jnp.concatenate([C1] * 16, axis=1)
                 + _bcast_cols(pre1, 0, 16) * jnp.concatenate([c2_blk] * 4, axis=1))
    c3row = jnp.sum(c3contrib, axis=0, keepdims=True)
    c1row = jnp.sum(C1, axis=0, keepdims=True)

    ed = lax.broadcasted_iota(jnp.int32, (D, D), 1)
    E = jnp.where(ed == idx_ref[:], sgn_ref[:], 0.0)
    baseline = jnp.dot(c3row, E, preferred_element_type=jnp.float32)

    inv_n = 1.0 / SEQ_LEN
    scaling = jnp.sum(c1row * cs_ref[:], axis=1, keepdims=True) * inv_n
    mods = jnp.dot(c1row, chm_ref[:], preferred_element_type=jnp.float32) * inv_n

    enhanced = (baseline * dw_ref[:] + bias_ref[:]) * scaling + mods
    hidden = jnp.maximum(
        jnp.dot(enhanced, w1t_ref[:], preferred_element_type=jnp.float32) + b1_ref[:], 0.0)
    out_ref[:] = jnp.dot(hidden, w2t_ref[:], preferred_element_type=jnp.float32) + b2_ref[:]


@functools.cache
def _sc_count_call():
    return pl.kernel(
        _sc_count,
        out_type=jax.ShapeDtypeStruct((NCHAINS * REC,), jnp.float32),
        mesh=plsc.VectorSubcoreMesh(core_axis_name="c", subcore_axis_name="s"),
        compiler_params=pltpu.CompilerParams(needs_layout_passes=False),
        scratch_types=[
            pltpu.VMEM((W_LEN,), jnp.int32),
            pltpu.VMEM((LANES * D,), jnp.float32),
            pltpu.VMEM((LANES * REC,), jnp.float32),
        ],
    )


def kernel(sequence, h_hash, s_signs, char_scales, dimension_weights, sketch_bias,
           char_hash_modifiers, W1, b1, W2, b2):
    partials = _sc_count_call()(sequence).reshape(NCHAINS, REC)

    idx64 = jnp.reshape(
        (h_hash[0][:, None, None] + h_hash[1][None, :, None] + h_hash[2][None, None, :]) % D,
        (D, 1)).astype(jnp.int32)
    sgn64 = jnp.reshape(
        s_signs[0][:, None, None] * s_signs[1][None, :, None] * s_signs[2][None, None, :],
        (D, 1))

    out = pl.pallas_call(
        _merge_kernel,
        out_shape=jax.ShapeDtypeStruct((1, D), jnp.float32),
    )(partials, idx64, sgn64,
      char_scales.reshape(1, ALPHA), char_hash_modifiers,
      dimension_weights.reshape(1, D), sketch_bias.reshape(1, D),
      W1.T, b1.reshape(1, D), W2.T, b2.reshape(1, D))
    return out.reshape(D)
